# baseline (speedup 1.0000x reference)
import functools

import jax
import jax.numpy as jnp
import numpy as np
from jax import lax
from jax.experimental import pallas as pl
from jax.experimental.pallas import tpu as pltpu
from jax.experimental.pallas import tpu_sc as plsc

NV, NO, D, H = 20000, 10000, 128, 4
DH = D // 2
NC, NS = 2, 16
NW = NC * NS
K = 128
CHUNK = NS * K


def _ceil_to(x, m):
    return ((x + m - 1) // m) * m


def _feats(x, wpairs, wquads):
    n = x.shape[0]
    nblk = (n + 127) // 128
    npair, nquad = len(wpairs), len(wquads)

    def body(*refs):
        x_ref = refs[0]
        w_refs = refs[1:1 + npair]
        q_refs = refs[1 + npair:1 + npair + nquad]
        o_refs = refs[1 + npair + nquad:]
        xb = x_ref[...]
        for i in range(npair):
            o_refs[i][0] = jnp.dot(xb, w_refs[i][0],
                                   preferred_element_type=jnp.float32)
        for i in range(nquad):
            o_refs[npair + i][...] = jnp.dot(xb, q_refs[i][...],
                                             preferred_element_type=jnp.float32)

    in_specs = [pl.BlockSpec((128, 128), lambda h, i: (i, 0))]
    in_specs += [pl.BlockSpec((1, 128, 256), lambda h, i: (h, 0, 0))] * npair
    in_specs += [pl.BlockSpec((128, 16), lambda h, i: (0, 0))] * nquad
    out_specs = [pl.BlockSpec((1, 128, 256), lambda h, i: (h, i, 0))] * npair
    out_specs += [pl.BlockSpec((128, 16), lambda h, i: (i, 0))] * nquad
    out_shape = [jax.ShapeDtypeStruct((2, n, 256), jnp.float32)] * npair
    out_shape += [jax.ShapeDtypeStruct((n, 16), jnp.float32)] * nquad
    res = pl.pallas_call(
        body, grid=(2, nblk), in_specs=in_specs, out_specs=out_specs,
        out_shape=out_shape)(x, *wpairs, *wquads)
    return res


def _sc_pass1_all(rels, tok):
    cfgs = []
    for (src, dst, t_as, t_ad, nd, e_real) in rels:
        ep = src.shape[0]
        ndp = _ceil_to(nd, NW)
        cfgs.append(dict(ep=ep, ndp=ndp, cps=ep // CHUNK, e_real=e_real,
                         zrows=ndp // NS, rows_per=ndp // NW))
    ndp_max = max(c['ndp'] for c in cfgs)
    nrel = len(rels)
    mesh = plsc.VectorSubcoreMesh(core_axis_name="c", subcore_axis_name="s")

    out_type = []
    for cfg in cfgs:
        out_type.append(jax.ShapeDtypeStruct((cfg['ep'], 16), jnp.float32))
        out_type.append(jax.ShapeDtypeStruct((cfg['ndp'], 16), jnp.float32))
    out_type.append(jax.ShapeDtypeStruct((8,), jnp.int32))

    HC = K // 2

    @functools.partial(
        pl.kernel, mesh=mesh,
        out_type=tuple(out_type),
        scratch_types=[
            pltpu.VMEM((8,), jnp.int32),
            pltpu.VMEM((2, HC), jnp.int32),
            pltpu.VMEM((2, HC), jnp.int32),
            pltpu.VMEM((2, HC, 16), jnp.float32),
            pltpu.VMEM((2, HC, 16), jnp.float32),
            pltpu.VMEM((2, HC, 16), jnp.float32),
            pltpu.VMEM((HC, 16), jnp.float32),
            pltpu.VMEM_SHARED((ndp_max, 16), jnp.float32),
            pltpu.SemaphoreType.DMA,
            pltpu.SemaphoreType.DMA,
            pltpu.SemaphoreType.DMA,
            pltpu.SemaphoreType.DMA,
        ],
        compiler_params=pltpu.CompilerParams(use_tc_tiling_on_sc=False,
                                             needs_layout_passes=False))
    def k(*refs):
        ins = refs[:4 * nrel]
        tok_h = refs[4 * nrel]
        outs = refs[4 * nrel + 1:4 * nrel + 1 + 2 * nrel]
        tok_o = refs[4 * nrel + 1 + 2 * nrel]
        (tokbuf, sidx, didx, sbuf, dbuf, exbuf, zbuf, den_acc,
         sems0, sems1, semd0, semd1) = refs[4 * nrel + 2 + 2 * nrel:]
        sems = [sems0, sems1]
        semd = [semd0, semd1]
        c = lax.axis_index("c")
        s = lax.axis_index("s")
        lanef = lax.iota(jnp.int32, 16).astype(jnp.float32)
        maskf = jnp.minimum(jnp.maximum(float(H) - lanef, 0.0), 1.0)
        zero = lanef * 0.0

        def zb(i, _):
            zbuf[i] = zero
            return 0
        lax.fori_loop(0, HC, zb, 0)

        for r, cfg in enumerate(cfgs):
            src_h, dst_h, tas_h, tad_h = ins[4 * r:4 * r + 4]
            ex_h, den_h = outs[2 * r:2 * r + 2]
            cps, e_real = cfg['cps'], cfg['e_real']
            zrows, rows_per = cfg['zrows'], cfg['rows_per']
            hcs = 2 * cps
            zc, zr = zrows // HC, zrows % HC
            for j in range(zc):
                pltpu.sync_copy(zbuf,
                                den_acc.at[pl.ds(s * zrows + j * HC, HC)])
            if zr:
                pltpu.sync_copy(zbuf.at[pl.ds(0, zr)],
                                den_acc.at[pl.ds(s * zrows + zc * HC, zr)])
            plsc.subcore_barrier()

            def start(t, b, src_h=src_h, dst_h=dst_h, tas_h=tas_h,
                      tad_h=tad_h, cps=cps):
                base = s * cps * K + t * HC
                pltpu.sync_copy(src_h.at[pl.ds(base, HC)], sidx.at[b])
                pltpu.sync_copy(dst_h.at[pl.ds(base, HC)], didx.at[b])
                pltpu.async_copy(tas_h.at[sidx.at[b]], sbuf.at[b], sems[b])
                pltpu.async_copy(tad_h.at[didx.at[b]], dbuf.at[b], semd[b])

            def finish(t, b, tas_h=tas_h, tad_h=tad_h, ex_h=ex_h,
                       cps=cps, e_real=e_real):
                base = s * cps * K + t * HC
                pltpu.make_async_copy(tas_h.at[sidx.at[b]], sbuf.at[b],
                                      sems[b]).wait()
                pltpu.make_async_copy(tad_h.at[didx.at[b]], dbuf.at[b],
                                      semd[b]).wait()

                def edge_body(i, _):
                    e = sbuf[b, i] + dbuf[b, i]
                    e = jnp.maximum(e, 0.2 * e)
                    ex = jnp.exp(e * maskf) * maskf
                    vf = jnp.minimum(jnp.maximum(e_real - base - i, 0),
                                     1).astype(jnp.float32)
                    exbuf[b, i] = ex * vf
                    return 0
                lax.fori_loop(0, HC, edge_body, 0)

                @pl.when(c == 0)
                def _():
                    pltpu.sync_copy(exbuf.at[b], ex_h.at[pl.ds(base, HC)])
                pltpu.sync_copy(exbuf.at[b], den_acc.at[didx.at[b]],
                                add=True)

            start(0, 0)

            def pair_body(tp, _, hcs=hcs, start=start, finish=finish):
                t0 = 2 * tp
                start(t0 + 1, 1)
                finish(t0, 0)

                @pl.when(t0 + 2 < hcs)
                def _():
                    start(t0 + 2, 0)
                finish(t0 + 1, 1)
                return 0
            lax.fori_loop(0, hcs // 2, pair_body, 0)
            plsc.subcore_barrier()
            row0 = (c * NS + s) * rows_per
            nwb = (rows_per + HC - 1) // HC
            for j in range(nwb):
                rows = min(HC, rows_per - j * HC)

                pltpu.sync_copy(den_acc.at[pl.ds(row0 + j * HC, rows)],
                                sbuf.at[0].at[pl.ds(0, rows)])

                def rcp_body(i, _):
                    sbuf[0, i] = 1.0 / (sbuf[0, i] + 1e-16)
                    return 0
                lax.fori_loop(0, rows, rcp_body, 0)
                pltpu.sync_copy(sbuf.at[0].at[pl.ds(0, rows)],
                                den_h.at[pl.ds(row0 + j * HC, rows)])
            plsc.subcore_barrier()

        @pl.when(jnp.logical_and(c == 0, s == 0))
        def _():
            pltpu.sync_copy(tok_h, tokbuf)
            pltpu.sync_copy(tokbuf, tok_o)

    flat_in = []
    for (src, dst, t_as, t_ad, nd, e_real) in rels:
        flat_in += [src, dst, t_as, t_ad]
    res = k(*flat_in, tok)
    pairs = [(res[2 * r], res[2 * r + 1]) for r in range(nrel)]
    return pairs, res[-1]


def _sc_pass2_all(rels, tok):
    cfgs = []
    for (src, dst, ex, den, hs2flat, ns, nd) in rels:
        ep = src.shape[0]
        ndp = den.shape[0]
        cfgs.append(dict(ep=ep, ndp=ndp, cps=ep // CHUNK, ns=ns,
                         zrows=ndp // NS))
    ndp_max = max(c['ndp'] for c in cfgs)
    nrel = len(rels)
    mesh = plsc.VectorSubcoreMesh(core_axis_name="c", subcore_axis_name="s")

    out_type = [jax.ShapeDtypeStruct((2 * c['ndp'], 64), jnp.float32)
                for c in cfgs]
    out_type.append(jax.ShapeDtypeStruct((8,), jnp.int32))

    HC = K // 2

    @functools.partial(
        pl.kernel, mesh=mesh,
        out_type=tuple(out_type),
        scratch_types=[
            pltpu.VMEM((8,), jnp.int32),
            pltpu.VMEM((2, HC), jnp.int32),
            pltpu.VMEM((2, HC), jnp.int32),
            pltpu.VMEM((2, HC, 256), jnp.float32),
            pltpu.VMEM((2, HC, 16), jnp.float32),
            pltpu.VMEM((2, HC, 16), jnp.float32),
            pltpu.VMEM((2, HC, 64), jnp.float32),
            pltpu.VMEM_SHARED((ndp_max, 64), jnp.float32),
            pltpu.SemaphoreType.DMA,
            pltpu.SemaphoreType.DMA,
            pltpu.SemaphoreType.DMA,
            pltpu.SemaphoreType.DMA,
        ],
        compiler_params=pltpu.CompilerParams(use_tc_tiling_on_sc=False,
                                             needs_layout_passes=False))
    def k(*refs):
        ins = refs[:5 * nrel]
        tok_h = refs[5 * nrel]
        outs = refs[5 * nrel + 1:5 * nrel + 1 + nrel]
        tok_o = refs[5 * nrel + 1 + nrel]
        (tokbuf, sidx, didx, hsbuf, exbuf, denbuf, msgbuf, acc,
         semg0, semg1, semd0, semd1) = refs[5 * nrel + 2 + nrel:]
        semg = [semg0, semg1]
        semd = [semd0, semd1]
        c = lax.axis_index("c")
        s = lax.axis_index("s")
        lanef = lax.iota(jnp.int32, 16).astype(jnp.float32)
        zero = lanef * 0.0

        for r, cfg in enumerate(cfgs):
            src_h, dst_h, ex_h, den_h, hs_h = ins[5 * r:5 * r + 5]
            out_h = outs[r]
            cps, ns, ndp, zrows = (cfg['cps'], cfg['ns'], cfg['ndp'],
                                   cfg['zrows'])
            hcs = 2 * cps
            zc, zr = zrows // HC, zrows % HC

            def zb(i, _):
                for j in range(4):
                    msgbuf[0, i, pl.ds(j * 16, 16)] = zero
                return 0
            lax.fori_loop(0, HC, zb, 0)
            for j in range(zc):
                pltpu.sync_copy(msgbuf.at[0],
                                acc.at[pl.ds(s * zrows + j * HC, HC)])
            if zr:
                pltpu.sync_copy(msgbuf.at[0].at[pl.ds(0, zr)],
                                acc.at[pl.ds(s * zrows + zc * HC, zr)])
            plsc.subcore_barrier()

            def start(t, b, src_h=src_h, dst_h=dst_h, ex_h=ex_h,
                      hs_h=hs_h, den_h=den_h, cps=cps, ns=ns):
                base = s * cps * K + t * HC
                pltpu.sync_copy(src_h.at[pl.ds(base, HC)], sidx.at[b])
                pltpu.sync_copy(dst_h.at[pl.ds(base, HC)], didx.at[b])

                def off_body(j, _):
                    sidx[b, pl.ds(j * 16, 16)] = (
                        sidx[b, pl.ds(j * 16, 16)] + c * ns)
                    return 0
                lax.fori_loop(0, HC // 16, off_body, 0)
                pltpu.async_copy(hs_h.at[sidx.at[b]], hsbuf.at[b], semg[b])
                pltpu.async_copy(den_h.at[didx.at[b]], denbuf.at[b], semd[b])
                pltpu.sync_copy(ex_h.at[pl.ds(base, HC)], exbuf.at[b])

            def finish(t, b, hs_h=hs_h, den_h=den_h):
                pltpu.make_async_copy(hs_h.at[sidx.at[b]], hsbuf.at[b],
                                      semg[b]).wait()
                pltpu.make_async_copy(den_h.at[didx.at[b]], denbuf.at[b],
                                      semd[b]).wait()

                def edge_body(i, _):
                    w = exbuf[b, i] * denbuf[b, i]
                    whs = [w[h] for h in range(H)]
                    for j in range(4):
                        m = whs[0] * hsbuf[b, i, pl.ds(j * 16, 16)]
                        for h in range(1, H):
                            m = m + whs[h] * hsbuf[b, i,
                                                   pl.ds(h * 64 + j * 16, 16)]
                        msgbuf[b, i, pl.ds(j * 16, 16)] = m
                    return 0
                lax.fori_loop(0, HC, edge_body, 0)
                pltpu.sync_copy(msgbuf.at[b], acc.at[didx.at[b]], add=True)

            start(0, 0)

            def pair_body(tp, _, hcs=hcs, start=start, finish=finish):
                t0 = 2 * tp
                start(t0 + 1, 1)
                finish(t0, 0)

                @pl.when(t0 + 2 < hcs)
                def _():
                    start(t0 + 2, 0)
                finish(t0 + 1, 1)
                return 0
            lax.fori_loop(0, hcs // 2, pair_body, 0)
            plsc.subcore_barrier()
            pltpu.sync_copy(acc.at[pl.ds(s * zrows, zrows)],
                            out_h.at[pl.ds(c * ndp + s * zrows, zrows)])
            plsc.subcore_barrier()

        @pl.when(jnp.logical_and(c == 0, s == 0))
        def _():
            pltpu.sync_copy(tok_h, tokbuf)
            pltpu.sync_copy(tokbuf, tok_o)

    flat_in = []
    for (src, dst, ex, den, hs2flat, ns, nd) in rels:
        flat_in += [src, dst, ex, den, hs2flat]
    res = k(*flat_in, tok)
    return list(res[:nrel]), res[-1]


def _combine(x, accs, pvec):
    n = x.shape[0]
    nblk = (n + 127) // 128
    nacc = len(accs)
    accs3 = [a.reshape(2, -1, 64) for a in accs]

    def body(*refs):
        x_ref = refs[0]
        al_refs = refs[1:1 + nacc]
        ar_refs = refs[1 + nacc:1 + 2 * nacc]
        p_ref = refs[1 + 2 * nacc]
        o_ref = refs[-1]
        hl = al_refs[0][0]
        hr = ar_refs[0][0]
        for i in range(1, nacc):
            hl = hl + al_refs[i][0]
            hr = hr + ar_refs[i][0]
        hm = jnp.concatenate([hl, hr], axis=1) * (1.0 / H) + p_ref[2:3, :]
        v = x_ref[...] + p_ref[3:4, :] * hm
        m = jnp.mean(v, axis=-1, keepdims=True)
        var = jnp.mean((v - m) ** 2, axis=-1, keepdims=True)
        y = (v - m) * lax.rsqrt(var + 1e-5) * p_ref[0:1, :] + p_ref[1:2, :]
        o_ref[...] = y

    in_specs = [pl.BlockSpec((128, 128), lambda i: (i, 0))]
    in_specs += [pl.BlockSpec((1, 128, 64), lambda i: (0, i, 0))] * nacc
    in_specs += [pl.BlockSpec((1, 128, 64), lambda i: (1, i, 0))] * nacc
    in_specs += [pl.BlockSpec((8, 128), lambda i: (0, 0))]
    args = [x] + accs3 + accs3 + [pvec]
    return pl.pallas_call(
        body, grid=(nblk,), in_specs=in_specs,
        out_specs=pl.BlockSpec((128, 128), lambda i: (i, 0)),
        out_shape=jax.ShapeDtypeStruct((n, 128), jnp.float32))(*args)


def _linout(x, w, b):
    n = x.shape[0]
    nblk = (n + 127) // 128
    wt = w.T
    bp = jnp.zeros((8, D), jnp.float32).at[0].set(b)

    def body(x_ref, w_ref, b_ref, o_ref):
        xb = x_ref[...]
        o_ref[...] = xb + jnp.dot(
            xb, w_ref[...], preferred_element_type=jnp.float32) + b_ref[0:1, :]

    return pl.pallas_call(
        body, grid=(nblk,),
        in_specs=[pl.BlockSpec((128, 128), lambda i: (i, 0)),
                  pl.BlockSpec((128, 128), lambda i: (0, 0)),
                  pl.BlockSpec((8, 128), lambda i: (0, 0))],
        out_specs=pl.BlockSpec((128, 128), lambda i: (i, 0)),
        out_shape=jax.ShapeDtypeStruct((n, 128), jnp.float32))(x, wt, bp)


def _prep_gat(p):
    w4 = p['W'].reshape(D, H, D)
    wl = w4[:, :, :DH].reshape(D, H * DH)
    wr = w4[:, :, DH:].reshape(D, H * DH)
    wpair = jnp.stack([wl, wr])
    q_as = jnp.pad(jnp.einsum('dhe,he->dh', w4, p['att_src']),
                   ((0, 0), (0, 16 - H)))
    q_ad = jnp.pad(jnp.einsum('dhe,he->dh', w4, p['att_dst']),
                   ((0, 0), (0, 16 - H)))
    return wpair, q_as, q_ad, p['b']


def _pad_edges(src, dst):
    e = src.shape[0]
    ep = _ceil_to(e, 2 * CHUNK)
    z = jnp.zeros((ep - e,), jnp.int32)
    return jnp.concatenate([src, z]), jnp.concatenate([dst, z]), e


def _pvec(ln, bias_sum, alpha):
    g, b = ln
    rows = [g, b, bias_sum, jnp.full((D,), alpha, jnp.float32)]
    rows += [jnp.zeros((D,), jnp.float32)] * 4
    return jnp.stack(rows)


def _layer(x_v, x_o, edges, pco, pcb, pnx, ln_v, ln_o, a_v, a_o, tok):
    (sco, dco, eco), (scb, dcb, ecb), (snx, dnx, enx) = edges
    wp_co, qas_co, qad_co, b_co = pco
    wp_cb, qas_cb, qad_cb, b_cb = pcb
    wp_nx, qas_nx, qad_nx, b_nx = pnx

    hs_co, hs_nx, t_as_co, t_ad_cb, t_as_nx, t_ad_nx = _feats(
        x_v, [wp_co, wp_nx], [qas_co, qad_cb, qas_nx, qad_nx])
    hs_cb, t_as_cb, t_ad_co = _feats(x_o, [wp_cb], [qas_cb, qad_co])

    p1, tok = _sc_pass1_all(
        [(sco, dco, t_as_co, t_ad_co, NO, eco),
         (scb, dcb, t_as_cb, t_ad_cb, NV, ecb),
         (snx, dnx, t_as_nx, t_ad_nx, NV, enx)], tok)
    (ex_co, den_co), (ex_cb, den_cb), (ex_nx, den_nx) = p1

    p2, tok = _sc_pass2_all(
        [(sco, dco, ex_co, den_co, hs_co.reshape(2 * NV, 256), NV, NO),
         (scb, dcb, ex_cb, den_cb, hs_cb.reshape(2 * NO, 256), NO, NV),
         (snx, dnx, ex_nx, den_nx, hs_nx.reshape(2 * NV, 256), NV, NV)], tok)
    acc_co, acc_cb, acc_nx = p2

    v1 = _combine(x_v, [acc_cb, acc_nx], _pvec(ln_v, b_cb + b_nx, a_v))
    o1 = _combine(x_o, [acc_co], _pvec(ln_o, b_co, a_o))
    return v1, o1, tok


def kernel(x_visit, x_occ, ei_contains, ei_contained_by, ei_next, params):
    p = params
    loop = jnp.arange(NV, dtype=jnp.int32)
    e_co = _pad_edges(ei_contains[0], ei_contains[1])
    e_cb = _pad_edges(ei_contained_by[0], ei_contained_by[1])
    e_nx = _pad_edges(jnp.concatenate([ei_next[0], loop]),
                      jnp.concatenate([ei_next[1], loop]))
    edges = (e_co, e_cb, e_nx)

    layer_params = []
    for l in (1, 2):
        layer_params.append((
            _prep_gat(p['c%d_co' % l]), _prep_gat(p['c%d_cb' % l]),
            _prep_gat(p['c%d_nx' % l]),
            p['ln_v%d' % l], p['ln_o%d' % l],
            p['alpha_v%d' % l], p['alpha_o%d' % l]))
    v2, o2 = x_visit, x_occ
    for ps in layer_params:
        pco, pcb, pnx, ln_v, ln_o, a_v, a_o = ps
        tok = jnp.zeros((8,), jnp.int32)
        v2, o2, tok = _layer(v2, o2, edges, pco, pcb, pnx,
                             ln_v, ln_o, a_v, a_o, tok)
    v_out = _linout(v2, p['lin_v_W'], p['lin_v_b'])
    o_out = _linout(o2, p['lin_o_W'], p['lin_o_b'])
    return jnp.concatenate([v_out, o_out], axis=0)


# speedup vs baseline: 1.0441x; 1.0441x over previous
import functools

import jax
import jax.numpy as jnp
import numpy as np
from jax import lax
from jax.experimental import pallas as pl
from jax.experimental.pallas import tpu as pltpu
from jax.experimental.pallas import tpu_sc as plsc

NV, NO, D, H = 20000, 10000, 128, 4
DH = D // 2
NC, NS = 2, 16
NW = NC * NS
K = 128
CHUNK = NS * K


def _ceil_to(x, m):
    return ((x + m - 1) // m) * m


def _feats(x, wpairs, wquads):
    n = x.shape[0]
    nblk = (n + 127) // 128
    npair, nquad = len(wpairs), len(wquads)

    def body(*refs):
        x_ref = refs[0]
        w_refs = refs[1:1 + npair]
        q_refs = refs[1 + npair:1 + npair + nquad]
        o_refs = refs[1 + npair + nquad:]
        xb = x_ref[...]
        for i in range(npair):
            o_refs[i][0] = jnp.dot(xb, w_refs[i][0],
                                   preferred_element_type=jnp.float32)
        for i in range(nquad):
            o_refs[npair + i][...] = jnp.dot(xb, q_refs[i][...],
                                             preferred_element_type=jnp.float32)

    in_specs = [pl.BlockSpec((128, 128), lambda h, i: (i, 0))]
    in_specs += [pl.BlockSpec((1, 128, 256), lambda h, i: (h, 0, 0))] * npair
    in_specs += [pl.BlockSpec((128, 16), lambda h, i: (0, 0))] * nquad
    out_specs = [pl.BlockSpec((1, 128, 256), lambda h, i: (h, i, 0))] * npair
    out_specs += [pl.BlockSpec((128, 16), lambda h, i: (i, 0))] * nquad
    out_shape = [jax.ShapeDtypeStruct((2, n, 256), jnp.float32)] * npair
    out_shape += [jax.ShapeDtypeStruct((n, 16), jnp.float32)] * nquad
    res = pl.pallas_call(
        body, grid=(2, nblk), in_specs=in_specs, out_specs=out_specs,
        out_shape=out_shape)(x, *wpairs, *wquads)
    return res


def _sc_pass1_all(rels, tok):
    cfgs = []
    for (src, dst, t_as, t_ad, nd, e_real) in rels:
        ep = src.shape[0]
        ndp = _ceil_to(nd, NW)
        cfgs.append(dict(ep=ep, ndp=ndp, cps=ep // CHUNK, e_real=e_real,
                         zrows=ndp // NS, rows_per=ndp // NW))
    ndp_max = max(c['ndp'] for c in cfgs)
    nrel = len(rels)
    mesh = plsc.VectorSubcoreMesh(core_axis_name="c", subcore_axis_name="s")

    out_type = []
    for cfg in cfgs:
        out_type.append(jax.ShapeDtypeStruct((cfg['ep'], 16), jnp.float32))
        out_type.append(jax.ShapeDtypeStruct((cfg['ndp'], 16), jnp.float32))
    out_type.append(jax.ShapeDtypeStruct((8,), jnp.int32))

    @functools.partial(
        pl.kernel, mesh=mesh,
        out_type=tuple(out_type),
        scratch_types=[
            pltpu.VMEM((8,), jnp.int32),
            pltpu.VMEM((K,), jnp.int32),
            pltpu.VMEM((K,), jnp.int32),
            pltpu.VMEM((K, 16), jnp.float32),
            pltpu.VMEM((K, 16), jnp.float32),
            pltpu.VMEM((K, 16), jnp.float32),
            pltpu.VMEM((K, 16), jnp.float32),
            pltpu.VMEM_SHARED((ndp_max, 16), jnp.float32),
            pltpu.SemaphoreType.DMA,
            pltpu.SemaphoreType.DMA,
        ],
        compiler_params=pltpu.CompilerParams(use_tc_tiling_on_sc=False,
                                             needs_layout_passes=False))
    def k(*refs):
        ins = refs[:4 * nrel]
        tok_h = refs[4 * nrel]
        outs = refs[4 * nrel + 1:4 * nrel + 1 + 2 * nrel]
        tok_o = refs[4 * nrel + 1 + 2 * nrel]
        (tokbuf, sidx, didx, sbuf, dbuf, exbuf, zbuf, den_acc,
         sem1, sem2) = refs[4 * nrel + 2 + 2 * nrel:]
        c = lax.axis_index("c")
        s = lax.axis_index("s")
        lanef = lax.iota(jnp.int32, 16).astype(jnp.float32)
        maskf = jnp.minimum(jnp.maximum(float(H) - lanef, 0.0), 1.0)
        zero = lanef * 0.0

        def zb(i, _):
            zbuf[i] = zero
            return 0
        lax.fori_loop(0, K, zb, 0)

        for r, cfg in enumerate(cfgs):
            src_h, dst_h, tas_h, tad_h = ins[4 * r:4 * r + 4]
            ex_h, den_h = outs[2 * r:2 * r + 2]
            cps, e_real = cfg['cps'], cfg['e_real']
            zrows, rows_per = cfg['zrows'], cfg['rows_per']
            zc, zr = zrows // K, zrows % K
            for j in range(zc):
                pltpu.sync_copy(zbuf, den_acc.at[pl.ds(s * zrows + j * K, K)])
            if zr:
                pltpu.sync_copy(zbuf.at[pl.ds(0, zr)],
                                den_acc.at[pl.ds(s * zrows + zc * K, zr)])
            plsc.subcore_barrier()

            def chunk_body(t, _, src_h=src_h, dst_h=dst_h, tas_h=tas_h,
                           tad_h=tad_h, ex_h=ex_h, cps=cps, e_real=e_real):
                base = (s * cps + t) * K
                pltpu.sync_copy(src_h.at[pl.ds(base, K)], sidx)
                pltpu.sync_copy(dst_h.at[pl.ds(base, K)], didx)
                cp1 = pltpu.async_copy(tas_h.at[sidx], sbuf, sem1)
                cp2 = pltpu.async_copy(tad_h.at[didx], dbuf, sem2)
                cp1.wait()
                cp2.wait()

                def edge_body(i, _):
                    e = sbuf[i] + dbuf[i]
                    e = jnp.maximum(e, 0.2 * e)
                    ex = jnp.exp(e * maskf) * maskf
                    vf = jnp.minimum(jnp.maximum(e_real - base - i, 0),
                                     1).astype(jnp.float32)
                    exbuf[i] = ex * vf
                    return 0
                lax.fori_loop(0, K, edge_body, 0)

                @pl.when(c == 0)
                def _():
                    pltpu.sync_copy(exbuf, ex_h.at[pl.ds(base, K)])
                pltpu.sync_copy(exbuf, den_acc.at[didx], add=True)
                return 0
            lax.fori_loop(0, cps, chunk_body, 0)
            plsc.subcore_barrier()
            row0 = (c * NS + s) * rows_per
            nwb = (rows_per + K - 1) // K
            for j in range(nwb):
                rows = min(K, rows_per - j * K)

                pltpu.sync_copy(den_acc.at[pl.ds(row0 + j * K, rows)],
                                sbuf.at[pl.ds(0, rows)])

                def rcp_body(i, _):
                    sbuf[i] = 1.0 / (sbuf[i] + 1e-16)
                    return 0
                lax.fori_loop(0, rows, rcp_body, 0)
                pltpu.sync_copy(sbuf.at[pl.ds(0, rows)],
                                den_h.at[pl.ds(row0 + j * K, rows)])
            plsc.subcore_barrier()

        @pl.when(jnp.logical_and(c == 0, s == 0))
        def _():
            pltpu.sync_copy(tok_h, tokbuf)
            pltpu.sync_copy(tokbuf, tok_o)

    flat_in = []
    for (src, dst, t_as, t_ad, nd, e_real) in rels:
        flat_in += [src, dst, t_as, t_ad]
    res = k(*flat_in, tok)
    pairs = [(res[2 * r], res[2 * r + 1]) for r in range(nrel)]
    return pairs, res[-1]


def _sc_pass2_all(rels, tok):
    cfgs = []
    for (src, dst, ex, den, hs2flat, ns, nd) in rels:
        ep = src.shape[0]
        ndp = den.shape[0]
        cfgs.append(dict(ep=ep, ndp=ndp, cps=ep // CHUNK, ns=ns,
                         zrows=ndp // NS))
    ndp_max = max(c['ndp'] for c in cfgs)
    nrel = len(rels)
    mesh = plsc.VectorSubcoreMesh(core_axis_name="c", subcore_axis_name="s")

    out_type = [jax.ShapeDtypeStruct((2 * c['ndp'], 64), jnp.float32)
                for c in cfgs]
    out_type.append(jax.ShapeDtypeStruct((8,), jnp.int32))

    HC = K // 2

    @functools.partial(
        pl.kernel, mesh=mesh,
        out_type=tuple(out_type),
        scratch_types=[
            pltpu.VMEM((8,), jnp.int32),
            pltpu.VMEM((2, HC), jnp.int32),
            pltpu.VMEM((2, HC), jnp.int32),
            pltpu.VMEM((2, HC, 256), jnp.float32),
            pltpu.VMEM((2, HC, 16), jnp.float32),
            pltpu.VMEM((2, HC, 16), jnp.float32),
            pltpu.VMEM((2, HC, 64), jnp.float32),
            pltpu.VMEM_SHARED((ndp_max, 64), jnp.float32),
            pltpu.SemaphoreType.DMA,
            pltpu.SemaphoreType.DMA,
            pltpu.SemaphoreType.DMA,
            pltpu.SemaphoreType.DMA,
        ],
        compiler_params=pltpu.CompilerParams(use_tc_tiling_on_sc=False,
                                             needs_layout_passes=False))
    def k(*refs):
        ins = refs[:5 * nrel]
        tok_h = refs[5 * nrel]
        outs = refs[5 * nrel + 1:5 * nrel + 1 + nrel]
        tok_o = refs[5 * nrel + 1 + nrel]
        (tokbuf, sidx, didx, hsbuf, exbuf, denbuf, msgbuf, acc,
         semg0, semg1, semd0, semd1) = refs[5 * nrel + 2 + nrel:]
        semg = [semg0, semg1]
        semd = [semd0, semd1]
        c = lax.axis_index("c")
        s = lax.axis_index("s")
        lanef = lax.iota(jnp.int32, 16).astype(jnp.float32)
        zero = lanef * 0.0

        for r, cfg in enumerate(cfgs):
            src_h, dst_h, ex_h, den_h, hs_h = ins[5 * r:5 * r + 5]
            out_h = outs[r]
            cps, ns, ndp, zrows = (cfg['cps'], cfg['ns'], cfg['ndp'],
                                   cfg['zrows'])
            hcs = 2 * cps
            zc, zr = zrows // HC, zrows % HC

            def zb(i, _):
                for j in range(4):
                    msgbuf[0, i, pl.ds(j * 16, 16)] = zero
                return 0
            lax.fori_loop(0, HC, zb, 0)
            for j in range(zc):
                pltpu.sync_copy(msgbuf.at[0],
                                acc.at[pl.ds(s * zrows + j * HC, HC)])
            if zr:
                pltpu.sync_copy(msgbuf.at[0].at[pl.ds(0, zr)],
                                acc.at[pl.ds(s * zrows + zc * HC, zr)])
            plsc.subcore_barrier()

            def start(t, b, src_h=src_h, dst_h=dst_h, ex_h=ex_h,
                      hs_h=hs_h, den_h=den_h, cps=cps, ns=ns):
                base = s * cps * K + t * HC
                pltpu.sync_copy(src_h.at[pl.ds(base, HC)], sidx.at[b])
                pltpu.sync_copy(dst_h.at[pl.ds(base, HC)], didx.at[b])

                def off_body(j, _):
                    sidx[b, pl.ds(j * 16, 16)] = (
                        sidx[b, pl.ds(j * 16, 16)] + c * ns)
                    return 0
                lax.fori_loop(0, HC // 16, off_body, 0)
                pltpu.async_copy(hs_h.at[sidx.at[b]], hsbuf.at[b], semg[b])
                pltpu.async_copy(den_h.at[didx.at[b]], denbuf.at[b], semd[b])
                pltpu.sync_copy(ex_h.at[pl.ds(base, HC)], exbuf.at[b])

            def finish(t, b, hs_h=hs_h, den_h=den_h):
                pltpu.make_async_copy(hs_h.at[sidx.at[b]], hsbuf.at[b],
                                      semg[b]).wait()
                pltpu.make_async_copy(den_h.at[didx.at[b]], denbuf.at[b],
                                      semd[b]).wait()

                def edge_body(i, _):
                    w = exbuf[b, i] * denbuf[b, i]
                    whs = [w[h] for h in range(H)]
                    for j in range(4):
                        m = whs[0] * hsbuf[b, i, pl.ds(j * 16, 16)]
                        for h in range(1, H):
                            m = m + whs[h] * hsbuf[b, i,
                                                   pl.ds(h * 64 + j * 16, 16)]
                        msgbuf[b, i, pl.ds(j * 16, 16)] = m
                    return 0
                lax.fori_loop(0, HC, edge_body, 0)
                pltpu.sync_copy(msgbuf.at[b], acc.at[didx.at[b]], add=True)

            start(0, 0)

            def pair_body(tp, _, hcs=hcs, start=start, finish=finish):
                t0 = 2 * tp
                start(t0 + 1, 1)
                finish(t0, 0)

                @pl.when(t0 + 2 < hcs)
                def _():
                    start(t0 + 2, 0)
                finish(t0 + 1, 1)
                return 0
            lax.fori_loop(0, hcs // 2, pair_body, 0)
            plsc.subcore_barrier()
            pltpu.sync_copy(acc.at[pl.ds(s * zrows, zrows)],
                            out_h.at[pl.ds(c * ndp + s * zrows, zrows)])
            plsc.subcore_barrier()

        @pl.when(jnp.logical_and(c == 0, s == 0))
        def _():
            pltpu.sync_copy(tok_h, tokbuf)
            pltpu.sync_copy(tokbuf, tok_o)

    flat_in = []
    for (src, dst, ex, den, hs2flat, ns, nd) in rels:
        flat_in += [src, dst, ex, den, hs2flat]
    res = k(*flat_in, tok)
    return list(res[:nrel]), res[-1]


def _combine(x, accs, pvec):
    n = x.shape[0]
    nblk = (n + 127) // 128
    nacc = len(accs)
    accs3 = [a.reshape(2, -1, 64) for a in accs]

    def body(*refs):
        x_ref = refs[0]
        al_refs = refs[1:1 + nacc]
        ar_refs = refs[1 + nacc:1 + 2 * nacc]
        p_ref = refs[1 + 2 * nacc]
        o_ref = refs[-1]
        hl = al_refs[0][0]
        hr = ar_refs[0][0]
        for i in range(1, nacc):
            hl = hl + al_refs[i][0]
            hr = hr + ar_refs[i][0]
        hm = jnp.concatenate([hl, hr], axis=1) * (1.0 / H) + p_ref[2:3, :]
        v = x_ref[...] + p_ref[3:4, :] * hm
        m = jnp.mean(v, axis=-1, keepdims=True)
        var = jnp.mean((v - m) ** 2, axis=-1, keepdims=True)
        y = (v - m) * lax.rsqrt(var + 1e-5) * p_ref[0:1, :] + p_ref[1:2, :]
        o_ref[...] = y

    in_specs = [pl.BlockSpec((128, 128), lambda i: (i, 0))]
    in_specs += [pl.BlockSpec((1, 128, 64), lambda i: (0, i, 0))] * nacc
    in_specs += [pl.BlockSpec((1, 128, 64), lambda i: (1, i, 0))] * nacc
    in_specs += [pl.BlockSpec((8, 128), lambda i: (0, 0))]
    args = [x] + accs3 + accs3 + [pvec]
    return pl.pallas_call(
        body, grid=(nblk,), in_specs=in_specs,
        out_specs=pl.BlockSpec((128, 128), lambda i: (i, 0)),
        out_shape=jax.ShapeDtypeStruct((n, 128), jnp.float32))(*args)


def _linout(x, w, b):
    n = x.shape[0]
    nblk = (n + 127) // 128
    wt = w.T
    bp = jnp.zeros((8, D), jnp.float32).at[0].set(b)

    def body(x_ref, w_ref, b_ref, o_ref):
        xb = x_ref[...]
        o_ref[...] = xb + jnp.dot(
            xb, w_ref[...], preferred_element_type=jnp.float32) + b_ref[0:1, :]

    return pl.pallas_call(
        body, grid=(nblk,),
        in_specs=[pl.BlockSpec((128, 128), lambda i: (i, 0)),
                  pl.BlockSpec((128, 128), lambda i: (0, 0)),
                  pl.BlockSpec((8, 128), lambda i: (0, 0))],
        out_specs=pl.BlockSpec((128, 128), lambda i: (i, 0)),
        out_shape=jax.ShapeDtypeStruct((n, 128), jnp.float32))(x, wt, bp)


def _prep_gat(p):
    w4 = p['W'].reshape(D, H, D)
    wl = w4[:, :, :DH].reshape(D, H * DH)
    wr = w4[:, :, DH:].reshape(D, H * DH)
    wpair = jnp.stack([wl, wr])
    q_as = jnp.pad(jnp.einsum('dhe,he->dh', w4, p['att_src']),
                   ((0, 0), (0, 16 - H)))
    q_ad = jnp.pad(jnp.einsum('dhe,he->dh', w4, p['att_dst']),
                   ((0, 0), (0, 16 - H)))
    return wpair, q_as, q_ad, p['b']


def _pad_edges(src, dst):
    e = src.shape[0]
    ep = _ceil_to(e, 2 * CHUNK)
    z = jnp.zeros((ep - e,), jnp.int32)
    return jnp.concatenate([src, z]), jnp.concatenate([dst, z]), e


def _pvec(ln, bias_sum, alpha):
    g, b = ln
    rows = [g, b, bias_sum, jnp.full((D,), alpha, jnp.float32)]
    rows += [jnp.zeros((D,), jnp.float32)] * 4
    return jnp.stack(rows)


def _layer(x_v, x_o, edges, pco, pcb, pnx, ln_v, ln_o, a_v, a_o, tok):
    (sco, dco, eco), (scb, dcb, ecb), (snx, dnx, enx) = edges
    wp_co, qas_co, qad_co, b_co = pco
    wp_cb, qas_cb, qad_cb, b_cb = pcb
    wp_nx, qas_nx, qad_nx, b_nx = pnx

    hs_co, hs_nx, t_as_co, t_ad_cb, t_as_nx, t_ad_nx = _feats(
        x_v, [wp_co, wp_nx], [qas_co, qad_cb, qas_nx, qad_nx])
    hs_cb, t_as_cb, t_ad_co = _feats(x_o, [wp_cb], [qas_cb, qad_co])

    p1, tok = _sc_pass1_all(
        [(sco, dco, t_as_co, t_ad_co, NO, eco),
         (scb, dcb, t_as_cb, t_ad_cb, NV, ecb),
         (snx, dnx, t_as_nx, t_ad_nx, NV, enx)], tok)
    (ex_co, den_co), (ex_cb, den_cb), (ex_nx, den_nx) = p1

    p2, tok = _sc_pass2_all(
        [(sco, dco, ex_co, den_co, hs_co.reshape(2 * NV, 256), NV, NO),
         (scb, dcb, ex_cb, den_cb, hs_cb.reshape(2 * NO, 256), NO, NV),
         (snx, dnx, ex_nx, den_nx, hs_nx.reshape(2 * NV, 256), NV, NV)], tok)
    acc_co, acc_cb, acc_nx = p2

    v1 = _combine(x_v, [acc_cb, acc_nx], _pvec(ln_v, b_cb + b_nx, a_v))
    o1 = _combine(x_o, [acc_co], _pvec(ln_o, b_co, a_o))
    return v1, o1, tok


def kernel(x_visit, x_occ, ei_contains, ei_contained_by, ei_next, params):
    p = params
    loop = jnp.arange(NV, dtype=jnp.int32)
    e_co = _pad_edges(ei_contains[0], ei_contains[1])
    e_cb = _pad_edges(ei_contained_by[0], ei_contained_by[1])
    e_nx = _pad_edges(jnp.concatenate([ei_next[0], loop]),
                      jnp.concatenate([ei_next[1], loop]))
    edges = (e_co, e_cb, e_nx)

    layer_params = []
    for l in (1, 2):
        layer_params.append((
            _prep_gat(p['c%d_co' % l]), _prep_gat(p['c%d_cb' % l]),
            _prep_gat(p['c%d_nx' % l]),
            p['ln_v%d' % l], p['ln_o%d' % l],
            p['alpha_v%d' % l], p['alpha_o%d' % l]))
    v2, o2 = x_visit, x_occ
    for ps in layer_params:
        pco, pcb, pnx, ln_v, ln_o, a_v, a_o = ps
        tok = jnp.zeros((8,), jnp.int32)
        v2, o2, tok = _layer(v2, o2, edges, pco, pcb, pnx,
                             ln_v, ln_o, a_v, a_o, tok)
    v_out = _linout(v2, p['lin_v_W'], p['lin_v_b'])
    o_out = _linout(o2, p['lin_o_W'], p['lin_o_b'])
    return jnp.concatenate([v_out, o_out], axis=0)


# speedup vs baseline: 1.0445x; 1.0003x over previous
import functools

import jax
import jax.numpy as jnp
import numpy as np
from jax import lax
from jax.experimental import pallas as pl
from jax.experimental.pallas import tpu as pltpu
from jax.experimental.pallas import tpu_sc as plsc

NV, NO, D, H = 20000, 10000, 128, 4
DH = D // 2
NC, NS = 2, 16
NW = NC * NS
K = 128
CHUNK = NS * K


def _ceil_to(x, m):
    return ((x + m - 1) // m) * m


def _feats(x, wpairs, wquads):
    n = x.shape[0]
    nblk = (n + 127) // 128
    npair, nquad = len(wpairs), len(wquads)

    def body(*refs):
        x_ref = refs[0]
        w_refs = refs[1:1 + npair]
        q_refs = refs[1 + npair:1 + npair + nquad]
        o_refs = refs[1 + npair + nquad:]
        xb = x_ref[...]
        for i in range(npair):
            o_refs[i][0] = jnp.dot(xb, w_refs[i][0],
                                   preferred_element_type=jnp.float32)
        for i in range(nquad):
            o_refs[npair + i][...] = jnp.dot(xb, q_refs[i][...],
                                             preferred_element_type=jnp.float32)

    in_specs = [pl.BlockSpec((128, 128), lambda h, i: (i, 0))]
    in_specs += [pl.BlockSpec((1, 128, 256), lambda h, i: (h, 0, 0))] * npair
    in_specs += [pl.BlockSpec((128, 16), lambda h, i: (0, 0))] * nquad
    out_specs = [pl.BlockSpec((1, 128, 256), lambda h, i: (h, i, 0))] * npair
    out_specs += [pl.BlockSpec((128, 16), lambda h, i: (i, 0))] * nquad
    out_shape = [jax.ShapeDtypeStruct((2, n, 256), jnp.float32)] * npair
    out_shape += [jax.ShapeDtypeStruct((n, 16), jnp.float32)] * nquad
    res = pl.pallas_call(
        body, grid=(2, nblk), in_specs=in_specs, out_specs=out_specs,
        out_shape=out_shape)(x, *wpairs, *wquads)
    return res


def _sc_pass1_all(rels, tok):
    cfgs = []
    for (src, dst, t_as, t_ad, nd, e_real) in rels:
        ep = src.shape[0]
        ndp = _ceil_to(nd, NW)
        cfgs.append(dict(ep=ep, ndp=ndp, cps=ep // CHUNK, e_real=e_real,
                         zrows=ndp // NS, rows_per=ndp // NW))
    ndp_max = max(c['ndp'] for c in cfgs)
    nrel = len(rels)
    mesh = plsc.VectorSubcoreMesh(core_axis_name="c", subcore_axis_name="s")

    out_type = []
    for cfg in cfgs:
        out_type.append(jax.ShapeDtypeStruct((cfg['ep'], 16), jnp.float32))
        out_type.append(jax.ShapeDtypeStruct((cfg['ndp'], 16), jnp.float32))
    out_type.append(jax.ShapeDtypeStruct((8,), jnp.int32))

    @functools.partial(
        pl.kernel, mesh=mesh,
        out_type=tuple(out_type),
        scratch_types=[
            pltpu.VMEM((8,), jnp.int32),
            pltpu.VMEM((K,), jnp.int32),
            pltpu.VMEM((K,), jnp.int32),
            pltpu.VMEM((K, 16), jnp.float32),
            pltpu.VMEM((K, 16), jnp.float32),
            pltpu.VMEM((K, 16), jnp.float32),
            pltpu.VMEM((K, 16), jnp.float32),
            pltpu.VMEM_SHARED((ndp_max, 16), jnp.float32),
            pltpu.SemaphoreType.DMA,
            pltpu.SemaphoreType.DMA,
        ],
        compiler_params=pltpu.CompilerParams(use_tc_tiling_on_sc=False,
                                             needs_layout_passes=False))
    def k(*refs):
        ins = refs[:4 * nrel]
        tok_h = refs[4 * nrel]
        outs = refs[4 * nrel + 1:4 * nrel + 1 + 2 * nrel]
        tok_o = refs[4 * nrel + 1 + 2 * nrel]
        (tokbuf, sidx, didx, sbuf, dbuf, exbuf, zbuf, den_acc,
         sem1, sem2) = refs[4 * nrel + 2 + 2 * nrel:]
        c = lax.axis_index("c")
        s = lax.axis_index("s")
        lanef = lax.iota(jnp.int32, 16).astype(jnp.float32)
        maskf = jnp.minimum(jnp.maximum(float(H) - lanef, 0.0), 1.0)
        zero = lanef * 0.0

        def zb(i, _):
            zbuf[i] = zero
            return 0
        lax.fori_loop(0, K, zb, 0)

        for r, cfg in enumerate(cfgs):
            src_h, dst_h, tas_h, tad_h = ins[4 * r:4 * r + 4]
            ex_h, den_h = outs[2 * r:2 * r + 2]
            cps, e_real = cfg['cps'], cfg['e_real']
            zrows, rows_per = cfg['zrows'], cfg['rows_per']
            zc, zr = zrows // K, zrows % K
            for j in range(zc):
                pltpu.sync_copy(zbuf, den_acc.at[pl.ds(s * zrows + j * K, K)])
            if zr:
                pltpu.sync_copy(zbuf.at[pl.ds(0, zr)],
                                den_acc.at[pl.ds(s * zrows + zc * K, zr)])
            plsc.subcore_barrier()

            def chunk_body(t, _, src_h=src_h, dst_h=dst_h, tas_h=tas_h,
                           tad_h=tad_h, ex_h=ex_h, cps=cps, e_real=e_real):
                base = (s * cps + t) * K
                pltpu.sync_copy(src_h.at[pl.ds(base, K)], sidx)
                pltpu.sync_copy(dst_h.at[pl.ds(base, K)], didx)
                cp1 = pltpu.async_copy(tas_h.at[sidx], sbuf, sem1)
                cp2 = pltpu.async_copy(tad_h.at[didx], dbuf, sem2)
                cp1.wait()
                cp2.wait()

                def edge_body(i, _):
                    e = sbuf[i] + dbuf[i]
                    e = jnp.maximum(e, 0.2 * e)
                    ex = jnp.exp(e * maskf) * maskf
                    vf = jnp.minimum(jnp.maximum(e_real - base - i, 0),
                                     1).astype(jnp.float32)
                    exbuf[i] = ex * vf
                    return 0
                lax.fori_loop(0, K, edge_body, 0)

                @pl.when(c == 0)
                def _():
                    pltpu.sync_copy(exbuf, ex_h.at[pl.ds(base, K)])
                pltpu.sync_copy(exbuf, den_acc.at[didx], add=True)
                return 0
            lax.fori_loop(0, cps, chunk_body, 0)
            plsc.subcore_barrier()
            row0 = (c * NS + s) * rows_per
            nwb = (rows_per + K - 1) // K
            for j in range(nwb):
                rows = min(K, rows_per - j * K)

                pltpu.sync_copy(den_acc.at[pl.ds(row0 + j * K, rows)],
                                sbuf.at[pl.ds(0, rows)])

                def rcp_body(i, _):
                    sbuf[i] = 1.0 / (sbuf[i] + 1e-16)
                    return 0
                lax.fori_loop(0, rows, rcp_body, 0)
                pltpu.sync_copy(sbuf.at[pl.ds(0, rows)],
                                den_h.at[pl.ds(row0 + j * K, rows)])
            plsc.subcore_barrier()

        @pl.when(jnp.logical_and(c == 0, s == 0))
        def _():
            pltpu.sync_copy(tok_h, tokbuf)
            pltpu.sync_copy(tokbuf, tok_o)

    flat_in = []
    for (src, dst, t_as, t_ad, nd, e_real) in rels:
        flat_in += [src, dst, t_as, t_ad]
    res = k(*flat_in, tok)
    pairs = [(res[2 * r], res[2 * r + 1]) for r in range(nrel)]
    return pairs, res[-1]


def _sc_pass2_all(rels, tok):
    cfgs = []
    for (src, dst, ex, den, hs2flat, ns, nd) in rels:
        ep = src.shape[0]
        ndp = den.shape[0]
        cfgs.append(dict(ep=ep, ndp=ndp, cps=ep // CHUNK, ns=ns,
                         zrows=ndp // NS))
    ndp_max = max(c['ndp'] for c in cfgs)
    nrel = len(rels)
    mesh = plsc.VectorSubcoreMesh(core_axis_name="c", subcore_axis_name="s")

    out_type = [jax.ShapeDtypeStruct((2 * c['ndp'], 64), jnp.float32)
                for c in cfgs]
    out_type.append(jax.ShapeDtypeStruct((8,), jnp.int32))

    HC = K // 2

    @functools.partial(
        pl.kernel, mesh=mesh,
        out_type=tuple(out_type),
        scratch_types=[
            pltpu.VMEM((8,), jnp.int32),
            pltpu.VMEM((2, HC), jnp.int32),
            pltpu.VMEM((2, HC), jnp.int32),
            pltpu.VMEM((2, HC, 256), jnp.float32),
            pltpu.VMEM((2, HC, 16), jnp.float32),
            pltpu.VMEM((2, HC, 16), jnp.float32),
            pltpu.VMEM((2, HC, 64), jnp.float32),
            pltpu.VMEM_SHARED((ndp_max, 64), jnp.float32),
            pltpu.SemaphoreType.DMA,
            pltpu.SemaphoreType.DMA,
            pltpu.SemaphoreType.DMA,
            pltpu.SemaphoreType.DMA,
        ],
        compiler_params=pltpu.CompilerParams(use_tc_tiling_on_sc=False,
                                             needs_layout_passes=False))
    def k(*refs):
        ins = refs[:5 * nrel]
        tok_h = refs[5 * nrel]
        outs = refs[5 * nrel + 1:5 * nrel + 1 + nrel]
        tok_o = refs[5 * nrel + 1 + nrel]
        (tokbuf, sidx, didx, hsbuf, exbuf, denbuf, msgbuf, acc,
         semg0, semg1, semd0, semd1) = refs[5 * nrel + 2 + nrel:]
        semg = [semg0, semg1]
        semd = [semd0, semd1]
        c = lax.axis_index("c")
        s = lax.axis_index("s")
        lanef = lax.iota(jnp.int32, 16).astype(jnp.float32)
        zero = lanef * 0.0

        for r, cfg in enumerate(cfgs):
            src_h, dst_h, ex_h, den_h, hs_h = ins[5 * r:5 * r + 5]
            out_h = outs[r]
            cps, ns, ndp, zrows = (cfg['cps'], cfg['ns'], cfg['ndp'],
                                   cfg['zrows'])
            hcs = 2 * cps
            zc, zr = zrows // HC, zrows % HC

            def zb(i, _):
                for j in range(4):
                    msgbuf[0, i, pl.ds(j * 16, 16)] = zero
                return 0
            lax.fori_loop(0, HC, zb, 0)
            for j in range(zc):
                pltpu.sync_copy(msgbuf.at[0],
                                acc.at[pl.ds(s * zrows + j * HC, HC)])
            if zr:
                pltpu.sync_copy(msgbuf.at[0].at[pl.ds(0, zr)],
                                acc.at[pl.ds(s * zrows + zc * HC, zr)])
            plsc.subcore_barrier()

            def start(t, b, src_h=src_h, dst_h=dst_h, ex_h=ex_h,
                      hs_h=hs_h, den_h=den_h, cps=cps, ns=ns):
                base = s * cps * K + t * HC
                pltpu.sync_copy(src_h.at[pl.ds(base, HC)], sidx.at[b])
                pltpu.sync_copy(dst_h.at[pl.ds(base, HC)], didx.at[b])

                def off_body(j, _):
                    sidx[b, pl.ds(j * 16, 16)] = (
                        sidx[b, pl.ds(j * 16, 16)] + c * ns)
                    return 0
                lax.fori_loop(0, HC // 16, off_body, 0)
                pltpu.async_copy(hs_h.at[sidx.at[b]], hsbuf.at[b], semg[b])
                pltpu.async_copy(den_h.at[didx.at[b]], denbuf.at[b], semd[b])
                pltpu.sync_copy(ex_h.at[pl.ds(base, HC)], exbuf.at[b])

            def finish(t, b, hs_h=hs_h, den_h=den_h):
                pltpu.make_async_copy(hs_h.at[sidx.at[b]], hsbuf.at[b],
                                      semg[b]).wait()
                pltpu.make_async_copy(den_h.at[didx.at[b]], denbuf.at[b],
                                      semd[b]).wait()

                def edge_body(i2, _):
                    for u in range(2):
                        i = i2 * 2 + u
                        w = exbuf[b, i] * denbuf[b, i]
                        whs = [w[h] for h in range(H)]
                        for j in range(4):
                            m = whs[0] * hsbuf[b, i, pl.ds(j * 16, 16)]
                            for h in range(1, H):
                                m = m + whs[h] * hsbuf[
                                    b, i, pl.ds(h * 64 + j * 16, 16)]
                            msgbuf[b, i, pl.ds(j * 16, 16)] = m
                    return 0
                lax.fori_loop(0, HC // 2, edge_body, 0)
                pltpu.sync_copy(msgbuf.at[b], acc.at[didx.at[b]], add=True)

            start(0, 0)

            def pair_body(tp, _, hcs=hcs, start=start, finish=finish):
                t0 = 2 * tp
                start(t0 + 1, 1)
                finish(t0, 0)

                @pl.when(t0 + 2 < hcs)
                def _():
                    start(t0 + 2, 0)
                finish(t0 + 1, 1)
                return 0
            lax.fori_loop(0, hcs // 2, pair_body, 0)
            plsc.subcore_barrier()
            pltpu.sync_copy(acc.at[pl.ds(s * zrows, zrows)],
                            out_h.at[pl.ds(c * ndp + s * zrows, zrows)])
            plsc.subcore_barrier()

        @pl.when(jnp.logical_and(c == 0, s == 0))
        def _():
            pltpu.sync_copy(tok_h, tokbuf)
            pltpu.sync_copy(tokbuf, tok_o)

    flat_in = []
    for (src, dst, ex, den, hs2flat, ns, nd) in rels:
        flat_in += [src, dst, ex, den, hs2flat]
    res = k(*flat_in, tok)
    return list(res[:nrel]), res[-1]


def _combine(x, accs, pvec):
    n = x.shape[0]
    nblk = (n + 127) // 128
    nacc = len(accs)
    accs3 = [a.reshape(2, -1, 64) for a in accs]

    def body(*refs):
        x_ref = refs[0]
        al_refs = refs[1:1 + nacc]
        ar_refs = refs[1 + nacc:1 + 2 * nacc]
        p_ref = refs[1 + 2 * nacc]
        o_ref = refs[-1]
        hl = al_refs[0][0]
        hr = ar_refs[0][0]
        for i in range(1, nacc):
            hl = hl + al_refs[i][0]
            hr = hr + ar_refs[i][0]
        hm = jnp.concatenate([hl, hr], axis=1) * (1.0 / H) + p_ref[2:3, :]
        v = x_ref[...] + p_ref[3:4, :] * hm
        m = jnp.mean(v, axis=-1, keepdims=True)
        var = jnp.mean((v - m) ** 2, axis=-1, keepdims=True)
        y = (v - m) * lax.rsqrt(var + 1e-5) * p_ref[0:1, :] + p_ref[1:2, :]
        o_ref[...] = y

    in_specs = [pl.BlockSpec((128, 128), lambda i: (i, 0))]
    in_specs += [pl.BlockSpec((1, 128, 64), lambda i: (0, i, 0))] * nacc
    in_specs += [pl.BlockSpec((1, 128, 64), lambda i: (1, i, 0))] * nacc
    in_specs += [pl.BlockSpec((8, 128), lambda i: (0, 0))]
    args = [x] + accs3 + accs3 + [pvec]
    return pl.pallas_call(
        body, grid=(nblk,), in_specs=in_specs,
        out_specs=pl.BlockSpec((128, 128), lambda i: (i, 0)),
        out_shape=jax.ShapeDtypeStruct((n, 128), jnp.float32))(*args)


def _linout(x, w, b):
    n = x.shape[0]
    nblk = (n + 127) // 128
    wt = w.T
    bp = jnp.zeros((8, D), jnp.float32).at[0].set(b)

    def body(x_ref, w_ref, b_ref, o_ref):
        xb = x_ref[...]
        o_ref[...] = xb + jnp.dot(
            xb, w_ref[...], preferred_element_type=jnp.float32) + b_ref[0:1, :]

    return pl.pallas_call(
        body, grid=(nblk,),
        in_specs=[pl.BlockSpec((128, 128), lambda i: (i, 0)),
                  pl.BlockSpec((128, 128), lambda i: (0, 0)),
                  pl.BlockSpec((8, 128), lambda i: (0, 0))],
        out_specs=pl.BlockSpec((128, 128), lambda i: (i, 0)),
        out_shape=jax.ShapeDtypeStruct((n, 128), jnp.float32))(x, wt, bp)


def _prep_gat(p):
    w4 = p['W'].reshape(D, H, D)
    wl = w4[:, :, :DH].reshape(D, H * DH)
    wr = w4[:, :, DH:].reshape(D, H * DH)
    wpair = jnp.stack([wl, wr])
    q_as = jnp.pad(jnp.einsum('dhe,he->dh', w4, p['att_src']),
                   ((0, 0), (0, 16 - H)))
    q_ad = jnp.pad(jnp.einsum('dhe,he->dh', w4, p['att_dst']),
                   ((0, 0), (0, 16 - H)))
    return wpair, q_as, q_ad, p['b']


def _pad_edges(src, dst):
    e = src.shape[0]
    ep = _ceil_to(e, 2 * CHUNK)
    z = jnp.zeros((ep - e,), jnp.int32)
    return jnp.concatenate([src, z]), jnp.concatenate([dst, z]), e


def _pvec(ln, bias_sum, alpha):
    g, b = ln
    rows = [g, b, bias_sum, jnp.full((D,), alpha, jnp.float32)]
    rows += [jnp.zeros((D,), jnp.float32)] * 4
    return jnp.stack(rows)


def _layer(x_v, x_o, edges, pco, pcb, pnx, ln_v, ln_o, a_v, a_o, tok):
    (sco, dco, eco), (scb, dcb, ecb), (snx, dnx, enx) = edges
    wp_co, qas_co, qad_co, b_co = pco
    wp_cb, qas_cb, qad_cb, b_cb = pcb
    wp_nx, qas_nx, qad_nx, b_nx = pnx

    hs_co, hs_nx, t_as_co, t_ad_cb, t_as_nx, t_ad_nx = _feats(
        x_v, [wp_co, wp_nx], [qas_co, qad_cb, qas_nx, qad_nx])
    hs_cb, t_as_cb, t_ad_co = _feats(x_o, [wp_cb], [qas_cb, qad_co])

    p1, tok = _sc_pass1_all(
        [(sco, dco, t_as_co, t_ad_co, NO, eco),
         (scb, dcb, t_as_cb, t_ad_cb, NV, ecb),
         (snx, dnx, t_as_nx, t_ad_nx, NV, enx)], tok)
    (ex_co, den_co), (ex_cb, den_cb), (ex_nx, den_nx) = p1

    p2, tok = _sc_pass2_all(
        [(sco, dco, ex_co, den_co, hs_co.reshape(2 * NV, 256), NV, NO),
         (scb, dcb, ex_cb, den_cb, hs_cb.reshape(2 * NO, 256), NO, NV),
         (snx, dnx, ex_nx, den_nx, hs_nx.reshape(2 * NV, 256), NV, NV)], tok)
    acc_co, acc_cb, acc_nx = p2

    v1 = _combine(x_v, [acc_cb, acc_nx], _pvec(ln_v, b_cb + b_nx, a_v))
    o1 = _combine(x_o, [acc_co], _pvec(ln_o, b_co, a_o))
    return v1, o1, tok


def kernel(x_visit, x_occ, ei_contains, ei_contained_by, ei_next, params):
    p = params
    loop = jnp.arange(NV, dtype=jnp.int32)
    e_co = _pad_edges(ei_contains[0], ei_contains[1])
    e_cb = _pad_edges(ei_contained_by[0], ei_contained_by[1])
    e_nx = _pad_edges(jnp.concatenate([ei_next[0], loop]),
                      jnp.concatenate([ei_next[1], loop]))
    edges = (e_co, e_cb, e_nx)

    layer_params = []
    for l in (1, 2):
        layer_params.append((
            _prep_gat(p['c%d_co' % l]), _prep_gat(p['c%d_cb' % l]),
            _prep_gat(p['c%d_nx' % l]),
            p['ln_v%d' % l], p['ln_o%d' % l],
            p['alpha_v%d' % l], p['alpha_o%d' % l]))
    v2, o2 = x_visit, x_occ
    for ps in layer_params:
        pco, pcb, pnx, ln_v, ln_o, a_v, a_o = ps
        tok = jnp.zeros((8,), jnp.int32)
        v2, o2, tok = _layer(v2, o2, edges, pco, pcb, pnx,
                             ln_v, ln_o, a_v, a_o, tok)
    v_out = _linout(v2, p['lin_v_W'], p['lin_v_b'])
    o_out = _linout(o2, p['lin_o_W'], p['lin_o_b'])
    return jnp.concatenate([v_out, o_out], axis=0)
